# baseline (speedup 1.0000x reference)
import functools

import jax
import jax.numpy as jnp
from jax import lax
from jax.experimental import pallas as pl
from jax.experimental.pallas import tpu as pltpu
from jax.experimental.pallas import tpu_sc as plsc

N = 10000
E = 320000
M = 2000
NC = 2
NS = 16
NW = NC * NS
CE = 4096
EP = 327680
FW = 4
NACC = 10240


def _make_sc_spmm(D, weighted):
    ng = D // FW
    np_ = NW // ng
    epp = EP // np_
    nchk = epp // CE
    mesh = plsc.VectorSubcoreMesh(core_axis_name="c", subcore_axis_name="s")

    @functools.partial(
        pl.kernel,
        out_type=jax.ShapeDtypeStruct((np_, D, NACC), jnp.float32),
        mesh=mesh,
        compiler_params=pltpu.CompilerParams(use_tc_tiling_on_sc=False,
                                             needs_layout_passes=False),
        scratch_types=[
            pltpu.VMEM((FW, NACC), jnp.float32),
            pltpu.VMEM((FW, NACC), jnp.float32),
            pltpu.VMEM((2, CE), jnp.int32),
            pltpu.VMEM((2, CE), jnp.int32),
            pltpu.VMEM((2, CE), jnp.float32),
            [pltpu.SemaphoreType.DMA] * 2,
            pltpu.SemaphoreType.DMA,
        ],
    )
    def k(ht_hbm, src_hbm, dst_hbm, w_hbm, out_hbm,
          ht_v, acc_v, srcb, dstb, wb, esems, hsem):
        cid = lax.axis_index("c")
        sid = lax.axis_index("s")
        wid = sid * NC + cid
        g = wid % ng
        p = wid // ng
        ebase = p * epp

        hdma = pltpu.async_copy(ht_hbm.at[pl.ds(g * FW, FW)], ht_v, hsem)

        def fire_chunk(c, j):
            off = ebase + c * CE
            pltpu.async_copy(src_hbm.at[pl.ds(off, CE)], srcb.at[j], esems[j])
            pltpu.async_copy(dst_hbm.at[pl.ds(off, CE)], dstb.at[j], esems[j])
            if weighted:
                pltpu.async_copy(w_hbm.at[pl.ds(off, CE)], wb.at[j], esems[j])

        def wait_chunk(j):
            pltpu.make_async_copy(src_hbm.at[pl.ds(0, CE)], srcb.at[j],
                                  esems[j]).wait()
            pltpu.make_async_copy(dst_hbm.at[pl.ds(0, CE)], dstb.at[j],
                                  esems[j]).wait()
            if weighted:
                pltpu.make_async_copy(w_hbm.at[pl.ds(0, CE)], wb.at[j],
                                      esems[j]).wait()

        fire_chunk(0, 0)

        zv = jnp.zeros((16,), jnp.float32)

        def zero_body(r, _):
            for f in range(FW):
                acc_v[f, pl.ds(r * 16, 16)] = zv
            return 0
        lax.fori_loop(0, NACC // 16, zero_body, 0)
        hdma.wait()

        def chunk_body(c, j):
            wait_chunk(j)

            @pl.when(c + 1 < nchk)
            def _():
                fire_chunk(c + 1, 1 - j)

            @plsc.parallel_loop(0, CE // 16, step=1, unroll=8)
            def _(i):
                sl = pl.ds(i * 16, 16)
                srcv = srcb[j, sl]
                dstv = dstb[j, sl]
                if weighted:
                    wv = wb[j, sl]
                for f in range(FW):
                    hv = plsc.load_gather(ht_v.at[f], [srcv])
                    if weighted:
                        hv = hv * wv
                    plsc.addupdate_scatter(acc_v.at[f], [dstv], hv)

        def iter_body(cc, _):
            chunk_body(cc * 2, 0)
            chunk_body(cc * 2 + 1, 1)
            return 0
        lax.fori_loop(0, nchk // 2, iter_body, 0)

        pltpu.sync_copy(acc_v, out_hbm.at[p, pl.ds(g * FW, FW)])

    return k


_sc_spmm_64w = _make_sc_spmm(64, True)
_sc_spmm_32w = _make_sc_spmm(32, True)
_sc_spmm_32u = _make_sc_spmm(32, False)


def _lin1_body(xt_ref, w1t_ref, b1_ref, o_ref):
    o_ref[...] = (jnp.dot(w1t_ref[...], xt_ref[...],
                          preferred_element_type=jnp.float32)
                  + b1_ref[...])


def _tc_lin1(xt, W1T, b1):
    return pl.pallas_call(
        _lin1_body,
        out_shape=jax.ShapeDtypeStruct((64, NACC), jnp.float32),
    )(xt, W1T, b1[:, None])


def _comb1_body(p_ref, w2t_ref, b2_ref, o_ref):
    h = jnp.maximum(p_ref[0] + p_ref[1], 0.0)
    o_ref[...] = (jnp.dot(w2t_ref[...], h,
                          preferred_element_type=jnp.float32)
                  + b2_ref[...])


def _tc_comb1(p, W2T, b2):
    return pl.pallas_call(
        _comb1_body,
        out_shape=jax.ShapeDtypeStruct((32, NACC), jnp.float32),
    )(p, W2T, b2[:, None])


def _comb2_body(p_ref, o_ref):
    o_ref[...] = jnp.maximum(p_ref[0] + p_ref[1] + p_ref[2] + p_ref[3], 0.0)


def _tc_comb2(p):
    return pl.pallas_call(
        _comb2_body,
        out_shape=jax.ShapeDtypeStruct((32, NACC), jnp.float32),
    )(p)


def _enew_body(adj_ref, ef_ref, we_ref, o_ref):
    t = jnp.dot(ef_ref[...], we_ref[...], preferred_element_type=jnp.float32)
    o_ref[...] = jnp.maximum(
        jnp.dot(adj_ref[...], t, preferred_element_type=jnp.float32), 0.0)


def _tc_enew(adj_e, edge_features, We):
    return pl.pallas_call(
        _enew_body,
        out_shape=jax.ShapeDtypeStruct((M, 32), jnp.float32),
    )(adj_e, edge_features, We)


_NBLK = 8
_BR = NACC // _NBLK


def _final_body(p_ref, t_ref, en_ref, wv_ref, wc1_ref, bc1_ref,
                wc2_ref, bc2_ref, o_ref):
    nft = (p_ref[0] + p_ref[1] + p_ref[2] + p_ref[3]) * (1.0 / float(E))
    nf = nft.T
    shared = jnp.maximum(
        jnp.dot(nf, wv_ref[...], preferred_element_type=jnp.float32)
        + jnp.dot(t_ref[...], en_ref[...], preferred_element_type=jnp.float32),
        0.0)
    l1 = jnp.maximum(
        jnp.dot(shared, wc1_ref[...], preferred_element_type=jnp.float32)
        + bc1_ref[...], 0.0)
    logits = (jnp.dot(l1, wc2_ref[...], preferred_element_type=jnp.float32)
              + bc2_ref[...])
    m = jnp.max(logits, axis=1, keepdims=True)
    ex = jnp.exp(logits - m)
    o_ref[...] = ex / jnp.sum(ex, axis=1, keepdims=True)


def _tc_final(p3, T, e_new, Wv, Wc1, bc1, Wc2, bc2):
    return pl.pallas_call(
        _final_body,
        grid=(_NBLK,),
        in_specs=[
            pl.BlockSpec((4, 32, _BR), lambda i: (0, 0, i)),
            pl.BlockSpec((_BR, M), lambda i: (i, 0)),
            pl.BlockSpec((M, 32), lambda i: (0, 0)),
            pl.BlockSpec((32, 32), lambda i: (0, 0)),
            pl.BlockSpec((32, 32), lambda i: (0, 0)),
            pl.BlockSpec((1, 32), lambda i: (0, 0)),
            pl.BlockSpec((32, 2), lambda i: (0, 0)),
            pl.BlockSpec((1, 2), lambda i: (0, 0)),
        ],
        out_specs=pl.BlockSpec((_BR, 2), lambda i: (i, 0)),
        out_shape=jax.ShapeDtypeStruct((N, 2), jnp.float32),
    )(p3, T, e_new, Wv, Wc1, bc1[None, :], Wc2, bc2[None, :])


def kernel(x, edge_index, edge_weight, edge_features, adj_e, T,
           W1, b1, W2, b2, We, Wv, Wc1, bc1, Wc2, bc2):
    pad = EP - E
    src = jnp.concatenate([edge_index[0], jnp.zeros((pad,), jnp.int32)])
    dst = jnp.concatenate([edge_index[1], jnp.full((pad,), N, jnp.int32)])
    w = jnp.concatenate([edge_weight, jnp.zeros((pad,), jnp.float32)])
    xt = jnp.concatenate(
        [x.T, jnp.zeros((x.shape[1], NACC - N), jnp.float32)], axis=1)

    g1 = _tc_lin1(xt, W1.T, b1)
    p1 = _sc_spmm_64w(g1, src, dst, w)
    g2 = _tc_comb1(p1, W2.T, b2)
    p2 = _sc_spmm_32w(g2, src, dst, w)
    h2 = _tc_comb2(p2)
    p3 = _sc_spmm_32u(h2, src, dst, w)
    e_new = _tc_enew(adj_e, edge_features, We)
    return _tc_final(p3, T, e_new, Wv, Wc1, bc1, Wc2, bc2)


# speedup vs baseline: 1.0172x; 1.0172x over previous
import functools

import jax
import jax.numpy as jnp
from jax import lax
from jax.experimental import pallas as pl
from jax.experimental.pallas import tpu as pltpu
from jax.experimental.pallas import tpu_sc as plsc

N = 10000
E = 320000
M = 2000
NC = 2
NS = 16
NW = NC * NS
CE = 4096
EP = 327680
FW = 4
NACC = 10240


def _make_sc_spmm(D, weighted):
    ng = D // FW
    np_ = NW // ng
    epp = EP // np_
    nchk = epp // CE
    mesh = plsc.VectorSubcoreMesh(core_axis_name="c", subcore_axis_name="s")

    @functools.partial(
        pl.kernel,
        out_type=jax.ShapeDtypeStruct((np_, D, NACC), jnp.float32),
        mesh=mesh,
        compiler_params=pltpu.CompilerParams(use_tc_tiling_on_sc=False,
                                             needs_layout_passes=False),
        scratch_types=[
            pltpu.VMEM((FW, NACC), jnp.float32),
            pltpu.VMEM((FW, NACC), jnp.float32),
            pltpu.VMEM((2, CE), jnp.int32),
            pltpu.VMEM((2, CE), jnp.int32),
            pltpu.VMEM((2, CE), jnp.float32),
            [pltpu.SemaphoreType.DMA] * 2,
            pltpu.SemaphoreType.DMA,
        ],
    )
    def k(ht_hbm, src_hbm, dst_hbm, w_hbm, out_hbm,
          ht_v, acc_v, srcb, dstb, wb, esems, hsem):
        cid = lax.axis_index("c")
        sid = lax.axis_index("s")
        wid = sid * NC + cid
        g = wid % ng
        p = wid // ng
        ebase = p * epp

        hdma = pltpu.async_copy(ht_hbm.at[pl.ds(g * FW, FW)], ht_v, hsem)

        def fire_chunk(c, j):
            off = ebase + c * CE
            pltpu.async_copy(src_hbm.at[pl.ds(off, CE)], srcb.at[j], esems[j])
            pltpu.async_copy(dst_hbm.at[pl.ds(off, CE)], dstb.at[j], esems[j])
            if weighted:
                pltpu.async_copy(w_hbm.at[pl.ds(off, CE)], wb.at[j], esems[j])

        def wait_chunk(j):
            pltpu.make_async_copy(src_hbm.at[pl.ds(0, CE)], srcb.at[j],
                                  esems[j]).wait()
            pltpu.make_async_copy(dst_hbm.at[pl.ds(0, CE)], dstb.at[j],
                                  esems[j]).wait()
            if weighted:
                pltpu.make_async_copy(w_hbm.at[pl.ds(0, CE)], wb.at[j],
                                      esems[j]).wait()

        fire_chunk(0, 0)

        zv = jnp.zeros((16,), jnp.float32)

        def zero_body(r, _):
            for f in range(FW):
                acc_v[f, pl.ds(r * 16, 16)] = zv
            return 0
        lax.fori_loop(0, NACC // 16, zero_body, 0)
        hdma.wait()

        def chunk_body(c, j):
            wait_chunk(j)

            @pl.when(c + 1 < nchk)
            def _():
                fire_chunk(c + 1, 1 - j)

            @plsc.parallel_loop(0, CE // 16, step=1, unroll=4)
            def _(i):
                sl = pl.ds(i * 16, 16)
                srcv = srcb[j, sl]
                dstv = dstb[j, sl]
                if weighted:
                    wv = wb[j, sl]
                for f in range(FW):
                    hv = plsc.load_gather(ht_v.at[f], [srcv])
                    if weighted:
                        hv = hv * wv
                    plsc.addupdate_scatter(acc_v.at[f], [dstv], hv)

        def iter_body(cc, _):
            chunk_body(cc * 2, 0)
            chunk_body(cc * 2 + 1, 1)
            return 0
        lax.fori_loop(0, nchk // 2, iter_body, 0)

        pltpu.sync_copy(acc_v, out_hbm.at[p, pl.ds(g * FW, FW)])

    return k


_sc_spmm_64w = _make_sc_spmm(64, True)
_sc_spmm_32w = _make_sc_spmm(32, True)
_sc_spmm_32u = _make_sc_spmm(32, False)


def _lin1_body(xt_ref, w1t_ref, b1_ref, o_ref):
    o_ref[...] = (jnp.dot(w1t_ref[...], xt_ref[...],
                          preferred_element_type=jnp.float32)
                  + b1_ref[...])


def _tc_lin1(xt, W1T, b1):
    return pl.pallas_call(
        _lin1_body,
        out_shape=jax.ShapeDtypeStruct((64, NACC), jnp.float32),
    )(xt, W1T, b1[:, None])


def _comb1_body(p_ref, w2t_ref, b2_ref, o_ref):
    h = jnp.maximum(p_ref[0] + p_ref[1], 0.0)
    o_ref[...] = (jnp.dot(w2t_ref[...], h,
                          preferred_element_type=jnp.float32)
                  + b2_ref[...])


def _tc_comb1(p, W2T, b2):
    return pl.pallas_call(
        _comb1_body,
        out_shape=jax.ShapeDtypeStruct((32, NACC), jnp.float32),
    )(p, W2T, b2[:, None])


def _comb2_body(p_ref, o_ref):
    o_ref[...] = jnp.maximum(p_ref[0] + p_ref[1] + p_ref[2] + p_ref[3], 0.0)


def _tc_comb2(p):
    return pl.pallas_call(
        _comb2_body,
        out_shape=jax.ShapeDtypeStruct((32, NACC), jnp.float32),
    )(p)


def _enew_body(adj_ref, ef_ref, we_ref, o_ref):
    t = jnp.dot(ef_ref[...], we_ref[...], preferred_element_type=jnp.float32)
    o_ref[...] = jnp.maximum(
        jnp.dot(adj_ref[...], t, preferred_element_type=jnp.float32), 0.0)


def _tc_enew(adj_e, edge_features, We):
    return pl.pallas_call(
        _enew_body,
        out_shape=jax.ShapeDtypeStruct((M, 32), jnp.float32),
    )(adj_e, edge_features, We)


_NBLK = 8
_BR = NACC // _NBLK


def _final_body(p_ref, t_ref, en_ref, wv_ref, wc1_ref, bc1_ref,
                wc2_ref, bc2_ref, o_ref):
    nft = (p_ref[0] + p_ref[1] + p_ref[2] + p_ref[3]) * (1.0 / float(E))
    nf = nft.T
    shared = jnp.maximum(
        jnp.dot(nf, wv_ref[...], preferred_element_type=jnp.float32)
        + jnp.dot(t_ref[...], en_ref[...], preferred_element_type=jnp.float32),
        0.0)
    l1 = jnp.maximum(
        jnp.dot(shared, wc1_ref[...], preferred_element_type=jnp.float32)
        + bc1_ref[...], 0.0)
    logits = (jnp.dot(l1, wc2_ref[...], preferred_element_type=jnp.float32)
              + bc2_ref[...])
    m = jnp.max(logits, axis=1, keepdims=True)
    ex = jnp.exp(logits - m)
    o_ref[...] = ex / jnp.sum(ex, axis=1, keepdims=True)


def _tc_final(p3, T, e_new, Wv, Wc1, bc1, Wc2, bc2):
    return pl.pallas_call(
        _final_body,
        grid=(_NBLK,),
        in_specs=[
            pl.BlockSpec((4, 32, _BR), lambda i: (0, 0, i)),
            pl.BlockSpec((_BR, M), lambda i: (i, 0)),
            pl.BlockSpec((M, 32), lambda i: (0, 0)),
            pl.BlockSpec((32, 32), lambda i: (0, 0)),
            pl.BlockSpec((32, 32), lambda i: (0, 0)),
            pl.BlockSpec((1, 32), lambda i: (0, 0)),
            pl.BlockSpec((32, 2), lambda i: (0, 0)),
            pl.BlockSpec((1, 2), lambda i: (0, 0)),
        ],
        out_specs=pl.BlockSpec((_BR, 2), lambda i: (i, 0)),
        out_shape=jax.ShapeDtypeStruct((N, 2), jnp.float32),
    )(p3, T, e_new, Wv, Wc1, bc1[None, :], Wc2, bc2[None, :])


def kernel(x, edge_index, edge_weight, edge_features, adj_e, T,
           W1, b1, W2, b2, We, Wv, Wc1, bc1, Wc2, bc2):
    pad = EP - E
    src = jnp.concatenate([edge_index[0], jnp.zeros((pad,), jnp.int32)])
    dst = jnp.concatenate([edge_index[1], jnp.full((pad,), N, jnp.int32)])
    w = jnp.concatenate([edge_weight, jnp.zeros((pad,), jnp.float32)])
    xt = jnp.concatenate(
        [x.T, jnp.zeros((x.shape[1], NACC - N), jnp.float32)], axis=1)

    g1 = _tc_lin1(xt, W1.T, b1)
    p1 = _sc_spmm_64w(g1, src, dst, w)
    g2 = _tc_comb1(p1, W2.T, b2)
    p2 = _sc_spmm_32w(g2, src, dst, w)
    h2 = _tc_comb2(p2)
    p3 = _sc_spmm_32u(h2, src, dst, w)
    e_new = _tc_enew(adj_e, edge_features, We)
    return _tc_final(p3, T, e_new, Wv, Wc1, bc1, Wc2, bc2)
